# Initial kernel scaffold; baseline (speedup 1.0000x reference)
import jax
import jax.numpy as jnp
from jax.experimental import pallas as pl


def kernel(x, edge_index, W_l1, b_l1, W_r1, W_l2, b_l2, W_r2):
    raise NotImplementedError("write your pallas kernel here")


# speedup vs baseline: 3.2191x; 3.2191x over previous
import functools

import jax
import jax.numpy as jnp
from jax import lax
from jax.experimental import pallas as pl
from jax.experimental.pallas import tpu as pltpu
from jax.experimental.pallas import tpu_sc as plsc

N = 10000
E = 320000
D = 128
NPAD = 10240
NC = 2
NS = 16
NW = NC * NS
CH = 80
E_PAD = 327680
CPW = E_PAD // NW // CH
IB = 16
NSTAGE = CPW // IB
ROWS_PER_TILE = NPAD // NS


def _sc_aggregate_body(feat, src_hbm, dst_hbm, agg_out,
                       ss0, ss1, sd0, sd1, r0, r1,
                       shared_agg, sg0, sg1, si):
    c = lax.axis_index("c")
    s = lax.axis_index("s")
    wid = s * NC + c

    zero16 = jnp.zeros((16,), jnp.float32)

    def _zrow_body(i, _):
        for k in range(D // 16):
            r0[i, pl.ds(k * 16, 16)] = zero16
        return 0
    lax.fori_loop(0, CH, _zrow_body, 0)

    slab0 = s * ROWS_PER_TILE
    for j in range(ROWS_PER_TILE // CH):
        pltpu.sync_copy(r0, shared_agg.at[pl.ds(slab0 + j * CH, CH)])
    plsc.subcore_barrier()

    def _stage_start(t, ss, sd):
        pltpu.async_copy(src_hbm.at[wid, pl.ds(t * IB, IB)], ss, si)
        pltpu.async_copy(dst_hbm.at[wid, pl.ds(t * IB, IB)], sd, si)

    def _stage_drain(ss, sd):
        pltpu.make_async_copy(src_hbm.at[0, pl.ds(0, IB)], ss, si).wait()
        pltpu.make_async_copy(dst_hbm.at[0, pl.ds(0, IB)], sd, si).wait()

    def _gather(ss, j, rbuf, sem):
        pltpu.async_copy(feat.at[ss.at[j]], rbuf, sem)

    def _drain(ss, j, rbuf, sem):
        pltpu.make_async_copy(feat.at[ss.at[j]], rbuf, sem).wait()

    def _scatter(sd, j, rbuf):
        pltpu.sync_copy(rbuf, shared_agg.at[sd.at[j]], add=True)

    _stage_start(0, ss0, sd0)
    _stage_drain(ss0, sd0)
    for t in range(NSTAGE):
        ss, sd = (ss0, sd0) if t % 2 == 0 else (ss1, sd1)
        if t + 1 < NSTAGE:
            nss, nsd = (ss1, sd1) if t % 2 == 0 else (ss0, sd0)
            _stage_start(t + 1, nss, nsd)
        _gather(ss, 0, r0, sg0)

        def _loop(j, _, ss=ss, sd=sd):
            a = 2 * j
            _gather(ss, a + 1, r1, sg1)
            _drain(ss, a, r0, sg0)
            _scatter(sd, a, r0)
            _gather(ss, a + 2, r0, sg0)
            _drain(ss, a + 1, r1, sg1)
            _scatter(sd, a + 1, r1)
            return 0
        lax.fori_loop(0, IB // 2 - 1, _loop, 0)
        _gather(ss, IB - 1, r1, sg1)
        _drain(ss, IB - 2, r0, sg0)
        _scatter(sd, IB - 2, r0)
        _drain(ss, IB - 1, r1, sg1)
        _scatter(sd, IB - 1, r1)
        if t + 1 < NSTAGE:
            _stage_drain(nss, nsd)

    plsc.subcore_barrier()
    pltpu.sync_copy(shared_agg.at[pl.ds(slab0, ROWS_PER_TILE)],
                    agg_out.at[c, pl.ds(slab0, ROWS_PER_TILE)])


def _make_sc_aggregate():
    mesh = plsc.VectorSubcoreMesh(core_axis_name="c", subcore_axis_name="s",
                                  num_cores=NC, num_subcores=NS)
    scratch = [
        pltpu.VMEM((IB, CH), jnp.int32),
        pltpu.VMEM((IB, CH), jnp.int32),
        pltpu.VMEM((IB, CH), jnp.int32),
        pltpu.VMEM((IB, CH), jnp.int32),
        pltpu.VMEM((CH, D), jnp.float32),
        pltpu.VMEM((CH, D), jnp.float32),
        pltpu.VMEM_SHARED((NPAD, D), jnp.float32),
        pltpu.SemaphoreType.DMA,
        pltpu.SemaphoreType.DMA,
        pltpu.SemaphoreType.DMA,
    ]
    return pl.kernel(
        _sc_aggregate_body,
        out_type=jax.ShapeDtypeStruct((NC, NPAD, D), jnp.float32),
        mesh=mesh,
        scratch_types=scratch,
        name="sage_sc_aggregate",
    )


def _sc_count_body(dst_hbm, cnt_out, dst_t, ones_r, zc, shared_cnt):
    c = lax.axis_index("c")
    s = lax.axis_index("s")
    wid = s * NC + c

    one16 = jnp.ones((16,), jnp.float32)
    zero16 = jnp.zeros((16,), jnp.float32)

    def _z_body(i, _):
        for k in range(D // 16):
            ones_r[i, pl.ds(k * 16, 16)] = one16
            zc[i, pl.ds(k * 16, 16)] = zero16
        return 0
    lax.fori_loop(0, CH, _z_body, 0)

    slab0 = s * ROWS_PER_TILE
    for j in range(ROWS_PER_TILE // CH):
        pltpu.sync_copy(zc, shared_cnt.at[pl.ds(slab0 + j * CH, CH)])
    pltpu.sync_copy(dst_hbm.at[wid], dst_t)
    plsc.subcore_barrier()

    def _loop(j, _):
        pltpu.sync_copy(ones_r, shared_cnt.at[dst_t.at[j]], add=True)
        return 0
    lax.fori_loop(0, CPW, _loop, 0)

    plsc.subcore_barrier()
    pltpu.sync_copy(shared_cnt.at[pl.ds(slab0, ROWS_PER_TILE)],
                    cnt_out.at[c, pl.ds(slab0, ROWS_PER_TILE)])


def _make_sc_count():
    mesh = plsc.VectorSubcoreMesh(core_axis_name="c", subcore_axis_name="s",
                                  num_cores=NC, num_subcores=NS)
    scratch = [
        pltpu.VMEM((CPW, CH), jnp.int32),
        pltpu.VMEM((CH, D), jnp.float32),
        pltpu.VMEM((CH, D), jnp.float32),
        pltpu.VMEM_SHARED((NPAD, D), jnp.float32),
    ]
    return pl.kernel(
        _sc_count_body,
        out_type=jax.ShapeDtypeStruct((NC, NPAD, D), jnp.float32),
        mesh=mesh,
        scratch_types=scratch,
        name="sage_sc_count",
    )


_BR = 1000


def _tc_layer1_body(x_b, a_b, c_b, wl, bl, wr, h_b):
    agg = a_b[0] + a_b[1]
    cnt = c_b[0, :, 0:1] + c_b[1, :, 0:1]
    mean = agg / jnp.maximum(cnt, 1.0)
    h = (jnp.dot(mean, wl[...], preferred_element_type=jnp.float32)
         + bl[...]
         + jnp.dot(x_b[...], wr[...], preferred_element_type=jnp.float32))
    h_b[...] = jnp.maximum(h, 0.0)


def _tc_layer2_body(h_in, a_b, c_b, wl, bl, wr, o_b):
    agg = a_b[0] + a_b[1]
    cnt = c_b[0, :, 0:1] + c_b[1, :, 0:1]
    mean = agg / jnp.maximum(cnt, 1.0)
    o = (jnp.dot(mean, wl[...], preferred_element_type=jnp.float32)
         + bl[...]
         + jnp.dot(h_in[...], wr[...], preferred_element_type=jnp.float32))
    m = jnp.max(o, axis=1, keepdims=True)
    lse = jnp.log(jnp.sum(jnp.exp(o - m), axis=1, keepdims=True)) + m
    o_b[...] = o - lse


def _tc_layer(body):
    in_specs = [
        pl.BlockSpec((_BR, D), lambda i: (i, 0)),
        pl.BlockSpec((NC, _BR, D), lambda i: (0, i, 0)),
        pl.BlockSpec((NC, _BR, D), lambda i: (0, i, 0)),
        pl.BlockSpec((D, D), lambda i: (0, 0)),
        pl.BlockSpec((1, D), lambda i: (0, 0)),
        pl.BlockSpec((D, D), lambda i: (0, 0)),
    ]
    return pl.pallas_call(
        body,
        grid=(N // _BR,),
        in_specs=in_specs,
        out_specs=pl.BlockSpec((_BR, D), lambda i: (i, 0)),
        out_shape=jax.ShapeDtypeStruct((N, D), jnp.float32),
    )


def kernel(x, edge_index, W_l1, b_l1, W_r1, W_l2, b_l2, W_r2):
    src = edge_index[0].astype(jnp.int32)
    dst = edge_index[1].astype(jnp.int32)
    pad = E_PAD - E
    src3 = jnp.concatenate(
        [src, jnp.zeros((pad,), jnp.int32)]).reshape(NW, CPW, CH)
    dst3 = jnp.concatenate(
        [dst, jnp.full((pad,), NPAD - 1, jnp.int32)]).reshape(NW, CPW, CH)
    bl1 = b_l1.reshape(1, D)
    bl2 = b_l2.reshape(1, D)

    sc_agg = _make_sc_aggregate()
    sc_cnt = _make_sc_count()
    tc1 = _tc_layer(_tc_layer1_body)
    tc2 = _tc_layer(_tc_layer2_body)

    cnt16 = sc_cnt(dst3)
    agg1 = sc_agg(x, src3, dst3)
    h = tc1(x, agg1, cnt16, W_l1, bl1, W_r1)
    agg2 = sc_agg(h, src3, dst3)
    out = tc2(h, agg2, cnt16, W_l2, bl2, W_r2)
    return (h, out)


# speedup vs baseline: 4.6033x; 1.4300x over previous
import jax
import jax.numpy as jnp
from jax import lax
from jax.experimental import pallas as pl
from jax.experimental.pallas import tpu as pltpu
from jax.experimental.pallas import tpu_sc as plsc

N = 10000
E = 320000
D = 128
DH = D // 2
NPAD = 10240
NC = 2
NS = 16
NW = NC * NS
CH = 128
E_PAD = 327680
CPT = E_PAD // NS // CH
IB = 32
NSTAGE = CPT // IB
ROWS_PER_TILE = NPAD // NS
NB = 4


def _sc_aggregate_body(feat2, src_hbm, dst_hbm, agg_out,
                       ss0, ss1, sd0, sd1, r0, r1, r2, r3,
                       shared_agg,
                       sg0, sg1, sg2, sg3, sc0, sc1, sc2, sc3, si):
    c = lax.axis_index("c")
    s = lax.axis_index("s")
    rbufs = (r0, r1, r2, r3)
    sgs = (sg0, sg1, sg2, sg3)
    scs = (sc0, sc1, sc2, sc3)

    zero16 = jnp.zeros((16,), jnp.float32)

    def _zrow_body(i, _):
        for k in range(DH // 16):
            r0[i, pl.ds(k * 16, 16)] = zero16
        return 0
    lax.fori_loop(0, CH, _zrow_body, 0)

    slab0 = s * ROWS_PER_TILE
    for j in range(ROWS_PER_TILE // CH):
        pltpu.sync_copy(r0, shared_agg.at[pl.ds(slab0 + j * CH, CH)])
    plsc.subcore_barrier()

    feat = feat2.at[c]

    def _stage_start(t, ss, sd):
        pltpu.async_copy(src_hbm.at[s, pl.ds(t * IB, IB)], ss, si)
        pltpu.async_copy(dst_hbm.at[s, pl.ds(t * IB, IB)], sd, si)

    def _stage_drain(ss, sd):
        pltpu.make_async_copy(src_hbm.at[0, pl.ds(0, IB)], ss, si).wait()
        pltpu.make_async_copy(dst_hbm.at[0, pl.ds(0, IB)], sd, si).wait()

    def _gather(ss, j, b):
        pltpu.async_copy(feat.at[ss.at[j]], rbufs[b], sgs[b])

    def _gdrain(ss, j, b):
        pltpu.make_async_copy(feat.at[ss.at[j]], rbufs[b], sgs[b]).wait()

    def _scatter(sd, j, b):
        pltpu.async_copy(rbufs[b], shared_agg.at[sd.at[j]], scs[b], add=True)

    def _sdrain(sd, j, b):
        pltpu.make_async_copy(rbufs[b], shared_agg.at[sd.at[j]], scs[b]).wait()

    _stage_start(0, ss0, sd0)
    _stage_drain(ss0, sd0)
    for t in range(NSTAGE):
        ss, sd = (ss0, sd0) if t % 2 == 0 else (ss1, sd1)
        if t + 1 < NSTAGE:
            nss, nsd = (ss1, sd1) if t % 2 == 0 else (ss0, sd0)
            _stage_start(t + 1, nss, nsd)
        _gather(ss, 0, 0)
        _gather(ss, 1, 1)
        _gdrain(ss, 0, 0)
        _scatter(sd, 0, 0)
        _gather(ss, 2, 2)
        _gdrain(ss, 1, 1)
        _scatter(sd, 1, 1)
        _gather(ss, 3, 3)

        def _loop(i, _, ss=ss, sd=sd):
            for u in range(NB):
                kk = NB * i + 2 + u
                b = (2 + u) % NB
                _sdrain(sd, kk - 2, u)
                _gdrain(ss, kk, b)
                _scatter(sd, kk, b)
                _gather(ss, kk + 2, u)
            return 0
        lax.fori_loop(0, (IB - 4) // NB, _loop, 0)
        k0 = IB - 2
        for kk in (k0, k0 + 1):
            b = kk % NB
            _sdrain(sd, kk - 2, (kk - 2) % NB)
            _gdrain(ss, kk, b)
            _scatter(sd, kk, b)
        _sdrain(sd, k0, k0 % NB)
        _sdrain(sd, k0 + 1, (k0 + 1) % NB)
        if t + 1 < NSTAGE:
            _stage_drain(nss, nsd)

    plsc.subcore_barrier()
    pltpu.sync_copy(shared_agg.at[pl.ds(slab0, ROWS_PER_TILE)],
                    agg_out.at[c, pl.ds(slab0, ROWS_PER_TILE)])


def _make_sc_aggregate():
    mesh = plsc.VectorSubcoreMesh(core_axis_name="c", subcore_axis_name="s",
                                  num_cores=NC, num_subcores=NS)
    scratch = [
        pltpu.VMEM((IB, CH), jnp.int32),
        pltpu.VMEM((IB, CH), jnp.int32),
        pltpu.VMEM((IB, CH), jnp.int32),
        pltpu.VMEM((IB, CH), jnp.int32),
        pltpu.VMEM((CH, DH), jnp.float32),
        pltpu.VMEM((CH, DH), jnp.float32),
        pltpu.VMEM((CH, DH), jnp.float32),
        pltpu.VMEM((CH, DH), jnp.float32),
        pltpu.VMEM_SHARED((NPAD, DH), jnp.float32),
    ] + [pltpu.SemaphoreType.DMA] * 9
    return pl.kernel(
        _sc_aggregate_body,
        out_type=jax.ShapeDtypeStruct((NC, NPAD, DH), jnp.float32),
        mesh=mesh,
        scratch_types=scratch,
        compiler_params=pltpu.CompilerParams(use_tc_tiling_on_sc=False),
        name="sage_sc_aggregate",
    )


CPW = E_PAD // NW // 80


def _sc_count_body(dst_hbm, cnt_out, dst_t, ones_r, zc, shared_cnt):
    c = lax.axis_index("c")
    s = lax.axis_index("s")
    wid = s * NC + c

    one16 = jnp.ones((16,), jnp.float32)
    zero16 = jnp.zeros((16,), jnp.float32)

    def _z_body(i, _):
        for k in range(D // 16):
            ones_r[i, pl.ds(k * 16, 16)] = one16
            zc[i, pl.ds(k * 16, 16)] = zero16
        return 0
    lax.fori_loop(0, 80, _z_body, 0)

    slab0 = s * ROWS_PER_TILE
    for j in range(ROWS_PER_TILE // 80):
        pltpu.sync_copy(zc, shared_cnt.at[pl.ds(slab0 + j * 80, 80)])
    pltpu.sync_copy(dst_hbm.at[wid], dst_t)
    plsc.subcore_barrier()

    def _loop(j, _):
        pltpu.sync_copy(ones_r, shared_cnt.at[dst_t.at[j]], add=True)
        return 0
    lax.fori_loop(0, CPW, _loop, 0)

    plsc.subcore_barrier()
    pltpu.sync_copy(shared_cnt.at[pl.ds(slab0, ROWS_PER_TILE)],
                    cnt_out.at[c, pl.ds(slab0, ROWS_PER_TILE)])


def _make_sc_count():
    mesh = plsc.VectorSubcoreMesh(core_axis_name="c", subcore_axis_name="s",
                                  num_cores=NC, num_subcores=NS)
    scratch = [
        pltpu.VMEM((CPW, 80), jnp.int32),
        pltpu.VMEM((80, D), jnp.float32),
        pltpu.VMEM((80, D), jnp.float32),
        pltpu.VMEM_SHARED((NPAD, D), jnp.float32),
    ]
    return pl.kernel(
        _sc_count_body,
        out_type=jax.ShapeDtypeStruct((NC, NPAD, D), jnp.float32),
        mesh=mesh,
        scratch_types=scratch,
        name="sage_sc_count",
    )


_BR = 1000


def _tc_layer1_body(x_b, a_b, c_b, wl, bl, wr, h_b, h2_b):
    agg = jnp.concatenate([a_b[0], a_b[1]], axis=1)
    cnt = c_b[0, :, 0:1] + c_b[1, :, 0:1]
    mean = agg / jnp.maximum(cnt, 1.0)
    h = (jnp.dot(mean, wl[...], preferred_element_type=jnp.float32)
         + bl[...]
         + jnp.dot(x_b[...], wr[...], preferred_element_type=jnp.float32))
    h = jnp.maximum(h, 0.0)
    h_b[...] = h
    h2_b[...] = jnp.stack([h[:, :DH], h[:, DH:]], axis=0)


def _tc_layer2_body(h_in, a_b, c_b, wl, bl, wr, o_b):
    agg = jnp.concatenate([a_b[0], a_b[1]], axis=1)
    cnt = c_b[0, :, 0:1] + c_b[1, :, 0:1]
    mean = agg / jnp.maximum(cnt, 1.0)
    o = (jnp.dot(mean, wl[...], preferred_element_type=jnp.float32)
         + bl[...]
         + jnp.dot(h_in[...], wr[...], preferred_element_type=jnp.float32))
    m = jnp.max(o, axis=1, keepdims=True)
    lse = jnp.log(jnp.sum(jnp.exp(o - m), axis=1, keepdims=True)) + m
    o_b[...] = o - lse


def _tc_in_specs():
    return [
        pl.BlockSpec((_BR, D), lambda i: (i, 0)),
        pl.BlockSpec((NC, _BR, DH), lambda i: (0, i, 0)),
        pl.BlockSpec((NC, _BR, D), lambda i: (0, i, 0)),
        pl.BlockSpec((D, D), lambda i: (0, 0)),
        pl.BlockSpec((1, D), lambda i: (0, 0)),
        pl.BlockSpec((D, D), lambda i: (0, 0)),
    ]


def _make_tc1():
    return pl.pallas_call(
        _tc_layer1_body,
        grid=(N // _BR,),
        in_specs=_tc_in_specs(),
        out_specs=[pl.BlockSpec((_BR, D), lambda i: (i, 0)),
                   pl.BlockSpec((NC, _BR, DH), lambda i: (0, i, 0))],
        out_shape=[jax.ShapeDtypeStruct((N, D), jnp.float32),
                   jax.ShapeDtypeStruct((NC, N, DH), jnp.float32)],
    )


def _make_tc2():
    return pl.pallas_call(
        _tc_layer2_body,
        grid=(N // _BR,),
        in_specs=_tc_in_specs(),
        out_specs=pl.BlockSpec((_BR, D), lambda i: (i, 0)),
        out_shape=jax.ShapeDtypeStruct((N, D), jnp.float32),
    )


def kernel(x, edge_index, W_l1, b_l1, W_r1, W_l2, b_l2, W_r2):
    src = edge_index[0].astype(jnp.int32)
    dst = edge_index[1].astype(jnp.int32)
    pad = E_PAD - E
    srcp = jnp.concatenate([src, jnp.zeros((pad,), jnp.int32)])
    dstp = jnp.concatenate([dst, jnp.full((pad,), NPAD - 1, jnp.int32)])
    src3 = srcp.reshape(NS, CPT, CH)
    dst3 = dstp.reshape(NS, CPT, CH)
    dst3c = dstp.reshape(NW, CPW, 80)
    x2 = jnp.swapaxes(x.reshape(N, NC, DH), 0, 1)
    bl1 = b_l1.reshape(1, D)
    bl2 = b_l2.reshape(1, D)

    sc_agg = _make_sc_aggregate()
    sc_cnt = _make_sc_count()
    tc1 = _make_tc1()
    tc2 = _make_tc2()

    cnt16 = sc_cnt(dst3c)
    agg1 = sc_agg(x2, src3, dst3)
    h, h2 = tc1(x, agg1, cnt16, W_l1, bl1, W_r1)
    agg2 = sc_agg(h2, src3, dst3)
    out = tc2(h, agg2, cnt16, W_l2, bl2, W_r2)
    return (h, out)
